# Initial kernel scaffold; baseline (speedup 1.0000x reference)
import jax
import jax.numpy as jnp
from jax.experimental import pallas as pl


def kernel(position, embedding):
    raise NotImplementedError("write your pallas kernel here")


# speedup vs baseline: 3.1720x; 3.1720x over previous
import functools

import jax
import jax.numpy as jnp
from jax import lax
from jax.experimental import pallas as pl
from jax.experimental.pallas import tpu as pltpu
from jax.experimental.pallas import tpu_sc as plsc

_DIM = 64
_MAX_LEN = 8192
_LANES = 16
_NUM_WORKERS = 32
_CHUNK = 128


def kernel(position, embedding):
    b, s = position.shape
    n = b * s
    per_worker = n // _NUM_WORKERS
    n_chunks = per_worker // _CHUNK

    idx_flat = position.reshape(n).astype(jnp.int32)

    mesh = plsc.VectorSubcoreMesh(core_axis_name="c", subcore_axis_name="s")

    @functools.partial(
        pl.kernel,
        mesh=mesh,
        out_type=jax.ShapeDtypeStruct((n, _DIM), jnp.float32),
        compiler_params=pltpu.CompilerParams(use_tc_tiling_on_sc=False),
        scratch_types=[
            pltpu.VMEM((_CHUNK,), jnp.int32),
            pltpu.VMEM((_CHUNK, _DIM), jnp.float32),
            pltpu.SemaphoreType.DMA,
        ],
    )
    def sc_gather(emb_hbm, idx_hbm, out_hbm, idx_v, rows_v, sem):
        wid = lax.axis_index("s") * 2 + lax.axis_index("c")
        base = wid * per_worker

        @pl.loop(0, n_chunks)
        def _(c):
            off = base + c * _CHUNK
            pltpu.sync_copy(idx_hbm.at[pl.ds(off, _CHUNK)], idx_v)

            @pl.loop(0, _CHUNK, step=_LANES)
            def _(i):
                v = idx_v[pl.ds(i, _LANES)] + (_MAX_LEN - 1)
                idx_v[pl.ds(i, _LANES)] = jnp.clip(v, 0, 2 * _MAX_LEN - 2)

            pltpu.async_copy(emb_hbm.at[idx_v], rows_v, sem).wait()
            pltpu.sync_copy(rows_v, out_hbm.at[pl.ds(off, _CHUNK)])

    out = sc_gather(embedding, idx_flat)
    return out.reshape(b, s, _DIM)


# speedup vs baseline: 4.0975x; 1.2918x over previous
import functools

import jax
import jax.numpy as jnp
from jax import lax
from jax.experimental import pallas as pl
from jax.experimental.pallas import tpu as pltpu
from jax.experimental.pallas import tpu_sc as plsc

_DIM = 64
_MAX_LEN = 8192
_LANES = 16
_NUM_WORKERS = 32
_CW = 128
_K = 4
_NBUF = 2


def kernel(position, embedding):
    b, s = position.shape
    n = b * s
    n_chunks = n // _CW
    chunks_per_worker = n_chunks // _NUM_WORKERS
    n_super = chunks_per_worker // _K

    idx2d = position.reshape(n_chunks, _CW).astype(jnp.int32)

    mesh = plsc.VectorSubcoreMesh(core_axis_name="c", subcore_axis_name="s")

    @functools.partial(
        pl.kernel,
        mesh=mesh,
        out_type=jax.ShapeDtypeStruct((n_chunks, _CW, _DIM), jnp.float32),
        compiler_params=pltpu.CompilerParams(use_tc_tiling_on_sc=False),
        scratch_types=[
            pltpu.VMEM((_NBUF, _K, _CW), jnp.int32),
            pltpu.VMEM((_NBUF, _K, _CW, _DIM), jnp.float32),
            pltpu.SemaphoreType.DMA,
            pltpu.SemaphoreType.DMA,
            pltpu.SemaphoreType.DMA,
            pltpu.SemaphoreType.DMA,
            pltpu.SemaphoreType.DMA,
            pltpu.SemaphoreType.DMA,
        ],
    )
    def sc_gather(emb_hbm, idx_hbm, out_hbm, idx_v, rows_v,
                  isem0, isem1, gsem0, gsem1, wsem0, wsem1):
        isem = (isem0, isem1)
        gsem = (gsem0, gsem1)
        wsem = (wsem0, wsem1)
        wid = lax.axis_index("s") * 2 + lax.axis_index("c")
        chunk_base = wid * chunks_per_worker

        for bb in range(_NBUF):
            pltpu.async_copy(
                idx_hbm.at[pl.ds(chunk_base + bb * _K, _K)], idx_v.at[bb],
                isem[bb])

        @pl.loop(0, n_super, step=_NBUF)
        def _(sc0):
            for bb in range(_NBUF):
                sidx = sc0 + bb
                c0 = chunk_base + sidx * _K

                pltpu.make_async_copy(
                    idx_hbm.at[pl.ds(c0, _K)], idx_v.at[bb], isem[bb]).wait()
                for j in range(_K):
                    @pl.loop(0, _CW, step=_LANES)
                    def _(i):
                        v = idx_v[bb, j, pl.ds(i, _LANES)] + (_MAX_LEN - 1)
                        idx_v[bb, j, pl.ds(i, _LANES)] = jnp.clip(
                            v, 0, 2 * _MAX_LEN - 2)

                @pl.when(sidx >= _NBUF)
                def _():
                    pltpu.make_async_copy(
                        rows_v.at[bb], out_hbm.at[pl.ds(c0 - _NBUF * _K, _K)],
                        wsem[bb]).wait()

                copies = [
                    pltpu.async_copy(
                        emb_hbm.at[idx_v.at[bb, j]], rows_v.at[bb, j],
                        gsem[bb])
                    for j in range(_K)
                ]
                for cp in copies:
                    cp.wait()

                @pl.when(sidx + _NBUF < n_super)
                def _():
                    pltpu.async_copy(
                        idx_hbm.at[pl.ds(c0 + _NBUF * _K, _K)], idx_v.at[bb],
                        isem[bb])

                pltpu.async_copy(
                    rows_v.at[bb], out_hbm.at[pl.ds(c0, _K)], wsem[bb])

        for bb in range(_NBUF):
            pltpu.make_async_copy(
                rows_v.at[bb], out_hbm.at[pl.ds(chunk_base, _K)],
                wsem[bb]).wait()

    out = sc_gather(embedding, idx2d)
    return out.reshape(b, s, _DIM)


# speedup vs baseline: 4.1492x; 1.0126x over previous
import functools

import jax
import jax.numpy as jnp
from jax import lax
from jax.experimental import pallas as pl
from jax.experimental.pallas import tpu as pltpu
from jax.experimental.pallas import tpu_sc as plsc

_DIM = 64
_MAX_LEN = 8192
_LANES = 16
_NUM_WORKERS = 32
_CW = 128
_K = 4
_NBUF = 2


def kernel(position, embedding):
    b, s = position.shape
    n = b * s
    n_chunks = n // _CW
    chunks_per_worker = n_chunks // _NUM_WORKERS
    n_super = chunks_per_worker // _K

    idx2d = position.reshape(n_chunks, _CW).astype(jnp.int32)

    mesh = plsc.VectorSubcoreMesh(core_axis_name="c", subcore_axis_name="s")

    @functools.partial(
        pl.kernel,
        mesh=mesh,
        out_type=jax.ShapeDtypeStruct((n_chunks, _CW, _DIM), jnp.float32),
        compiler_params=pltpu.CompilerParams(use_tc_tiling_on_sc=False),
        scratch_types=[
            pltpu.VMEM((_NBUF, _K, _CW), jnp.int32),
            pltpu.VMEM((_NBUF, _K, _CW, _DIM), jnp.float32),
            pltpu.SemaphoreType.DMA,
            pltpu.SemaphoreType.DMA,
            pltpu.SemaphoreType.DMA,
            pltpu.SemaphoreType.DMA,
            pltpu.SemaphoreType.DMA,
            pltpu.SemaphoreType.DMA,
        ],
    )
    def sc_gather(emb_hbm, idx_hbm, out_hbm, idx_v, rows_v,
                  isem0, isem1, gsem0, gsem1, wsem0, wsem1):
        isem = (isem0, isem1)
        gsem = (gsem0, gsem1)
        wsem = (wsem0, wsem1)
        wid = lax.axis_index("s") * 2 + lax.axis_index("c")
        chunk_base = wid * chunks_per_worker

        def fire_gathers(bb):
            for j in range(_K):
                pltpu.async_copy(
                    emb_hbm.at[idx_v.at[bb, j]], rows_v.at[bb, j], gsem[bb])

        def drain_gathers(bb):
            for j in range(_K):
                pltpu.make_async_copy(
                    emb_hbm.at[idx_v.at[bb, j]], rows_v.at[bb, j],
                    gsem[bb]).wait()

        pltpu.async_copy(idx_hbm.at[pl.ds(chunk_base, _K)], idx_v.at[0],
                         isem[0])

        @pl.loop(0, n_super, step=_NBUF)
        def _(sc0):
            for bb in range(_NBUF):
                ob = 1 - bb
                sidx = sc0 + bb
                c0 = chunk_base + sidx * _K

                pltpu.make_async_copy(
                    idx_hbm.at[pl.ds(c0, _K)], idx_v.at[bb], isem[bb]).wait()
                for j in range(_K):
                    @pl.loop(0, _CW, step=_LANES)
                    def _(i):
                        v = idx_v[bb, j, pl.ds(i, _LANES)] + (_MAX_LEN - 1)
                        idx_v[bb, j, pl.ds(i, _LANES)] = jnp.clip(
                            v, 0, 2 * _MAX_LEN - 2)

                @pl.when(sidx >= _NBUF)
                def _():
                    pltpu.make_async_copy(
                        rows_v.at[bb], out_hbm.at[pl.ds(c0 - _NBUF * _K, _K)],
                        wsem[bb]).wait()

                fire_gathers(bb)

                @pl.when(sidx >= 1)
                def _():
                    drain_gathers(ob)
                    pltpu.async_copy(
                        rows_v.at[ob], out_hbm.at[pl.ds(c0 - _K, _K)],
                        wsem[ob])

                @pl.when(sidx + 1 < n_super)
                def _():
                    pltpu.async_copy(
                        idx_hbm.at[pl.ds(c0 + _K, _K)], idx_v.at[ob],
                        isem[ob])

        last = n_super - 1
        drain_gathers(1)
        pltpu.async_copy(
            rows_v.at[1], out_hbm.at[pl.ds(chunk_base + last * _K, _K)],
            wsem[1])
        for bb in range(_NBUF):
            pltpu.make_async_copy(
                rows_v.at[bb], out_hbm.at[pl.ds(chunk_base, _K)],
                wsem[bb]).wait()

    out = sc_gather(embedding, idx2d)
    return out.reshape(b, s, _DIM)
